# Initial kernel scaffold; baseline (speedup 1.0000x reference)
import jax
import jax.numpy as jnp
from jax.experimental import pallas as pl


def kernel(x, edge_index, batch, W1, b1, g1, be1, W2, b2, Wp1, bp1, gp, bep, Wp2, bp2):
    raise NotImplementedError("write your pallas kernel here")


# speedup vs baseline: 3.4060x; 3.4060x over previous
import functools

import jax
import jax.numpy as jnp
from jax import lax
from jax.experimental import pallas as pl
from jax.experimental.pallas import tpu as pltpu
from jax.experimental.pallas import tpu_sc as plsc

N = 10000
H = 128
E = 320000
G = 64
CL = 4
LY = 2

NTILE = 16
NW = 16
RPT = 632
NP = NTILE * RPT
TRASH = N
K = 40
NFIRE = 3
GEDGE = NFIRE * K
NPAIR = -(-(E // NW) // (2 * GEDGE))
NGRP = 2 * NPAIR
EPW = NGRP * GEDGE
EPAD = NW * EPW

_mesh = plsc.VectorSubcoreMesh(core_axis_name="c", subcore_axis_name="s",
                               num_cores=1)


@functools.partial(
    pl.kernel,
    out_type=jax.ShapeDtypeStruct((1, NP, H), jnp.float32),
    mesh=_mesh,
    scratch_types=[
        pltpu.VMEM((NFIRE, K), jnp.int32),
        pltpu.VMEM((NFIRE, K), jnp.int32),
        pltpu.VMEM((NFIRE, K), jnp.int32),
        pltpu.VMEM((NFIRE, K), jnp.int32),
        pltpu.VMEM((NFIRE, K, H), jnp.float32),
        pltpu.VMEM((NFIRE, K, H), jnp.float32),
        pltpu.VMEM_SHARED((NP, H), jnp.float32),
        pltpu.SemaphoreType.DMA,
        pltpu.SemaphoreType.DMA,
    ],
)
def _sc_seg_sum(h_hbm, src_hbm, dst_hbm, z_hbm, out_hbm, sa_src, sa_dst,
                sb_src, sb_dst, ra, rb, agg, sem_a, sem_b):
    ci = lax.axis_index("c")
    si = lax.axis_index("s")
    wid = ci * NTILE + si

    pltpu.sync_copy(z_hbm, agg.at[pl.ds(si * RPT, RPT)])
    plsc.subcore_barrier()

    def load_idx(g, s_src, s_dst):
        pltpu.sync_copy(src_hbm.at[wid, g], s_src)
        pltpu.sync_copy(dst_hbm.at[wid, g], s_dst)

    def fire(s_src, rows, sem):
        for j in range(NFIRE):
            pltpu.async_copy(h_hbm.at[s_src.at[j]], rows.at[j], sem)

    def drain_scatter(s_src, s_dst, rows, sem):
        for j in range(NFIRE):
            pltpu.make_async_copy(h_hbm.at[s_src.at[j]], rows.at[j],
                                  sem).wait()
        for j in range(NFIRE):
            pltpu.sync_copy(rows.at[j], agg.at[s_dst.at[j]], add=True)

    load_idx(0, sa_src, sa_dst)
    fire(sa_src, ra, sem_a)

    def pair(i, carry):
        load_idx(2 * i + 1, sb_src, sb_dst)
        fire(sb_src, rb, sem_b)
        drain_scatter(sa_src, sa_dst, ra, sem_a)

        @pl.when(i + 1 < NPAIR)
        def _():
            load_idx(2 * i + 2, sa_src, sa_dst)
            fire(sa_src, ra, sem_a)

        drain_scatter(sb_src, sb_dst, rb, sem_b)
        return carry

    lax.fori_loop(0, NPAIR, pair, 0)
    plsc.subcore_barrier()
    pltpu.sync_copy(agg.at[pl.ds(si * RPT, RPT)],
                    out_hbm.at[ci, pl.ds(si * RPT, RPT)])


def _bf16_dot(a, b):
    return jnp.dot(a, b, preferred_element_type=jnp.float32)


def _mlp_body(part_ref, h_ref, lab_ref, cval_ref, W1_ref, b1_ref, g1_ref,
              be1_ref, W2_ref, b2_ref, out_ref):
    h = h_ref[...]
    agg = part_ref[0, :N, :] + h
    maskf = (lab_ref[...] == cval_ref[...]).astype(jnp.float32)
    h1 = _bf16_dot(agg, W1_ref[...]) + b1_ref[...]
    cnt = jnp.maximum(jnp.sum(maskf), 1.0)
    s1 = jnp.sum(h1 * maskf, axis=0, keepdims=True)
    s2 = jnp.sum(h1 * h1 * maskf, axis=0, keepdims=True)
    m = s1 / cnt
    v = s2 / cnt - m * m
    h1n = g1_ref[...] * (h1 - m) / jnp.sqrt(v + 1e-5) + be1_ref[...]
    h1n = jnp.maximum(h1n, 0.0)
    new = _bf16_dot(h1n, W2_ref[...]) + b2_ref[...]
    out_ref[...] = jnp.where(maskf > 0.0, new, h)


_mlp_update = pl.pallas_call(
    _mlp_body,
    out_shape=jax.ShapeDtypeStruct((N, H), jnp.float32),
)


def _pool_body(h_ref, b_ref, out_ref):
    onehot = (b_ref[...] == lax.broadcasted_iota(jnp.int32, (1, G), 1))
    out_ref[...] = lax.dot_general(
        onehot.astype(jnp.float32), h_ref[...],
        (((0,), (0,)), ((), ())),
        preferred_element_type=jnp.float32,
        precision=lax.Precision.HIGHEST)


_pool = pl.pallas_call(
    _pool_body,
    out_shape=jax.ShapeDtypeStruct((G, H), jnp.float32),
)


def _head_body(p_ref, Wp1_ref, bp1_ref, gp_ref, bep_ref, Wp2_ref, bp2_ref,
               out_ref):
    p = p_ref[...]
    h1 = _bf16_dot(p, Wp1_ref[...]) + bp1_ref[...]
    m = jnp.mean(h1, axis=0, keepdims=True)
    v = jnp.mean((h1 - m) ** 2, axis=0, keepdims=True)
    h1 = gp_ref[...] * (h1 - m) / jnp.sqrt(v + 1e-5) + bep_ref[...]
    h1 = jnp.maximum(h1, 0.0)
    out_ref[...] = _bf16_dot(h1, Wp2_ref[...]) + bp2_ref[...]


_head = pl.pallas_call(
    _head_body,
    out_shape=jax.ShapeDtypeStruct((G, H), jnp.float32),
)


def kernel(x, edge_index, batch, W1, b1, g1, be1, W2, b2, Wp1, bp1, gp, bep,
           Wp2, bp2):
    labels = x[:, 0:1]
    h = x[:, 1:]
    src_pad = jnp.concatenate(
        [edge_index[0], jnp.zeros((EPAD - E,), edge_index.dtype)])
    dst_pad = jnp.concatenate(
        [edge_index[1], jnp.full((EPAD - E,), TRASH, edge_index.dtype)])
    src2 = src_pad.astype(jnp.int32).reshape(NW, NGRP, NFIRE, K)
    dst2 = dst_pad.astype(jnp.int32).reshape(NW, NGRP, NFIRE, K)
    zblk = jnp.zeros((RPT, H), jnp.float32)
    batch2 = batch.astype(jnp.int32).reshape(N, 1)

    pools = []
    for t in range(LY):
        for c in range(CL):
            i = t * CL + c
            part = _sc_seg_sum(h, src2, dst2, zblk)
            cval = jnp.full((1, 1), float(c), jnp.float32)
            h = _mlp_update(part, h, labels, cval,
                            W1[i], b1[i].reshape(1, H), g1[i].reshape(1, H),
                            be1[i].reshape(1, H), W2[i], b2[i].reshape(1, H))
        pools.append(_pool(h, batch2))

    p = jnp.concatenate(pools, axis=1)
    return _head(p, Wp1, bp1.reshape(1, H), gp.reshape(1, H),
                 bep.reshape(1, H), Wp2, bp2.reshape(1, H))
